# Initial kernel scaffold; baseline (speedup 1.0000x reference)
import jax
import jax.numpy as jnp
from jax.experimental import pallas as pl


def kernel(vert_feat_in, vert_mask_in, vert_element_oh, adj_oh, atom_subsets, atom_subsets_peaks, ln_g, ln_b, gru_w_ih, gru_w_hh, gru_b_ih, gru_b_hh, l1_w, l1_b, l2a_w, l2a_b, l2b_w, l2b_b, ln2_g, ln2_b, score_w, score_b):
    raise NotImplementedError("write your pallas kernel here")


# speedup vs baseline: 2.0645x; 2.0645x over previous
import jax
import jax.numpy as jnp
import numpy as np
from jax.experimental import pallas as pl

_FORMULA_OH_SIZES = [20, 20, 20, 20, 20]
_SPECT_BIN_N = 512


def _ln(x, g, b, eps=1e-5):
    m = jnp.mean(x, axis=-1, keepdims=True)
    v = jnp.mean((x - m) ** 2, axis=-1, keepdims=True)
    return (x - m) * jax.lax.rsqrt(v + eps) * g + b


def _batch_kernel(
    x_ref,
    mask_ref,
    elem_ref,
    subs_ref,
    mass_ref,
    inten_ref,
    ln_g_ref, ln_b_ref,
    wih_ref,
    whh_ref,
    bih_ref, bhh_ref,
    l1w_ref,
    l1b_ref,
    l2aw_ref, l2ab_ref,
    l2bw_ref, l2bb_ref,
    ln2g_ref, ln2b_ref,
    sw_ref,
    sb_ref,
    spect_ref,
    probs_ref,
):
    S, A = subs_ref.shape[1], subs_ref.shape[2]
    G = x_ref.shape[2]
    M = mass_ref.shape[2]
    E = elem_ref.shape[2]
    F = 20 * E

    x = x_ref[0]
    mask = mask_ref[0]
    subs_raw = subs_ref[0]
    subs_f = subs_raw * mask

    masked_x = x * mask.reshape(A, 1)
    swvs = jnp.dot(subs_f, masked_x, preferred_element_type=jnp.float32)
    size = jnp.sum(subs_f, axis=1, keepdims=True) + 0.0001
    mean = swvs / size
    h = _ln(mean, ln_g_ref[...], ln_b_ref[...])

    col = jax.lax.broadcasted_iota(jnp.int32, (E, F), 1)
    row = jax.lax.broadcasted_iota(jnp.int32, (E, F), 0)
    seg = (col // 20 == row).astype(jnp.float32)
    p_mat = jnp.dot(elem_ref[0], seg, preferred_element_type=jnp.float32)
    cx = jnp.dot(subs_raw, p_mat, preferred_element_type=jnp.float32)
    thresh = jnp.clip(cx, 0.0, 19.0)
    colmod = (jax.lax.broadcasted_iota(jnp.int32, (S, F), 1) % 20).astype(jnp.float32)
    pf = (colmod >= thresh).astype(jnp.float32)

    gi = jnp.dot(pf, wih_ref[...], preferred_element_type=jnp.float32) + bih_ref[...]
    gh = jnp.dot(h, whh_ref[...], preferred_element_type=jnp.float32) + bhh_ref[...]
    i_r, i_z, i_n = gi[:, :G], gi[:, G:2 * G], gi[:, 2 * G:]
    h_r, h_z, h_n = gh[:, :G], gh[:, G:2 * G], gh[:, 2 * G:]
    r = jax.nn.sigmoid(i_r + h_r)
    z = jax.nn.sigmoid(i_z + h_z)
    n = jnp.tanh(i_n + r * h_n)
    hn = (1.0 - z) * n + z * h

    x1 = jax.nn.relu(jnp.dot(hn, l1w_ref[...], preferred_element_type=jnp.float32) + l1b_ref[...])
    x2 = jax.nn.relu(jnp.dot(x1, l2aw_ref[...], preferred_element_type=jnp.float32) + l2ab_ref[...])
    x2 = jax.nn.relu(jnp.dot(x2, l2bw_ref[...], preferred_element_type=jnp.float32) + l2bb_ref[...])
    xn = _ln(x2, ln2g_ref[...], ln2b_ref[...])
    scores = jnp.sum(xn * sw_ref[...], axis=1, keepdims=True) + sb_ref[0, 0]

    smax = jnp.max(scores, axis=0, keepdims=True)
    e = jnp.exp(scores - smax)
    probs = e / jnp.sum(e, axis=0, keepdims=True)
    probs_ref[0] = probs

    bins = jnp.clip(jnp.round(mass_ref[0]), 0.0, float(_SPECT_BIN_N - 1))
    contrib = inten_ref[0] * probs
    lane = jax.lax.broadcasted_iota(jnp.int32, (S, _SPECT_BIN_N), 1).astype(jnp.float32)
    acc = jnp.zeros((S, _SPECT_BIN_N), jnp.float32)
    for m in range(M):
        onehot = (bins[:, m:m + 1] == lane).astype(jnp.float32)
        acc = acc + contrib[:, m:m + 1] * onehot
    spect_ref[0] = jnp.sum(acc, axis=0, keepdims=True)


def kernel(vert_feat_in, vert_mask_in, vert_element_oh, adj_oh, atom_subsets,
           atom_subsets_peaks, ln_g, ln_b, gru_w_ih, gru_w_hh, gru_b_ih,
           gru_b_hh, l1_w, l1_b, l2a_w, l2a_b, l2b_w, l2b_b, ln2_g, ln2_b,
           score_w, score_b):
    B, A, GF0, HW = vert_feat_in.shape
    G = GF0 * HW
    S = atom_subsets.shape[1]
    M = atom_subsets_peaks.shape[2]
    E = vert_element_oh.shape[2]
    F = int(np.sum(_FORMULA_OH_SIZES))
    D = l1_w.shape[0]

    x = vert_feat_in.reshape(B, A, G)
    mask3 = vert_mask_in.reshape(B, 1, A)
    elem_f = vert_element_oh.astype(jnp.float32)
    subs_f = atom_subsets.astype(jnp.float32)
    mass = atom_subsets_peaks[..., 0]
    inten = atom_subsets_peaks[..., 1]

    row = lambda v: v.reshape(1, -1)

    def bspec(shape, mapped=True):
        if mapped:
            return pl.BlockSpec(shape, lambda b: (b,) + (0,) * (len(shape) - 1))
        return pl.BlockSpec(shape, lambda b: (0,) * len(shape))

    in_specs = [
        bspec((1, A, G)),
        bspec((1, 1, A)),
        bspec((1, A, E)),
        bspec((1, S, A)),
        bspec((1, S, M)),
        bspec((1, S, M)),
        bspec((1, G), mapped=False),
        bspec((1, G), mapped=False),
        bspec((F, 3 * G), mapped=False),
        bspec((G, 3 * G), mapped=False),
        bspec((1, 3 * G), mapped=False),
        bspec((1, 3 * G), mapped=False),
        bspec((G, D), mapped=False),
        bspec((1, D), mapped=False),
        bspec((D, D), mapped=False),
        bspec((1, D), mapped=False),
        bspec((D, D), mapped=False),
        bspec((1, D), mapped=False),
        bspec((1, D), mapped=False),
        bspec((1, D), mapped=False),
        bspec((1, D), mapped=False),
        bspec((1, 1), mapped=False),
    ]
    out_specs = [
        bspec((1, 1, _SPECT_BIN_N)),
        bspec((1, S, 1)),
    ]
    spect3, probs3 = pl.pallas_call(
        _batch_kernel,
        grid=(B,),
        in_specs=in_specs,
        out_specs=out_specs,
        out_shape=[
            jax.ShapeDtypeStruct((B, 1, _SPECT_BIN_N), jnp.float32),
            jax.ShapeDtypeStruct((B, S, 1), jnp.float32),
        ],
    )(
        x, mask3, elem_f, subs_f, mass, inten,
        row(ln_g), row(ln_b),
        gru_w_ih.T, gru_w_hh.T, row(gru_b_ih), row(gru_b_hh),
        l1_w.T, row(l1_b), l2a_w.T, row(l2a_b), l2b_w.T, row(l2b_b),
        row(ln2_g), row(ln2_b), score_w, score_b.reshape(1, 1),
    )
    return spect3.reshape(B, _SPECT_BIN_N), probs3.reshape(B, S)


# speedup vs baseline: 2.1631x; 1.0478x over previous
import jax
import jax.numpy as jnp
import numpy as np
from jax.experimental import pallas as pl

_FORMULA_OH_SIZES = [20, 20, 20, 20, 20]
_SPECT_BIN_N = 512


def _dot_t(x, w):
    return jax.lax.dot_general(
        x, w, (((1,), (1,)), ((), ())), preferred_element_type=jnp.float32)


def _ln(x, g, b, eps=1e-5):
    m = jnp.mean(x, axis=-1, keepdims=True)
    v = jnp.mean((x - m) ** 2, axis=-1, keepdims=True)
    return (x - m) * jax.lax.rsqrt(v + eps) * g + b


def _batch_kernel(
    x_ref,
    mask_ref,
    elem_ref,
    subs_ref,
    mass_ref,
    inten_ref,
    ln_g_ref, ln_b_ref,
    wih_ref,
    whh_ref,
    bih_ref, bhh_ref,
    l1w_ref,
    l1b_ref,
    l2aw_ref, l2ab_ref,
    l2bw_ref, l2bb_ref,
    ln2g_ref, ln2b_ref,
    sw_ref,
    sb_ref,
    spect_ref,
    probs_ref,
):
    S, A = subs_ref.shape[1], subs_ref.shape[2]
    G = x_ref.shape[2]
    M = mass_ref.shape[2]
    E = elem_ref.shape[2]
    F = 20 * E

    x = x_ref[0]
    mask = mask_ref[0]
    subs_raw = subs_ref[0]
    subs_f = subs_raw * mask

    masked_x = x * mask.reshape(A, 1)
    swvs = jnp.dot(subs_f, masked_x, preferred_element_type=jnp.float32)
    size = jnp.sum(subs_f, axis=1, keepdims=True) + 0.0001
    mean = swvs / size
    h = _ln(mean, ln_g_ref[...], ln_b_ref[...])

    col = jax.lax.broadcasted_iota(jnp.int32, (E, F), 1)
    row = jax.lax.broadcasted_iota(jnp.int32, (E, F), 0)
    seg = (col // 20 == row).astype(jnp.float32)
    p_mat = jnp.dot(elem_ref[0], seg, preferred_element_type=jnp.float32)
    cx = jnp.dot(subs_raw, p_mat, preferred_element_type=jnp.float32)
    thresh = jnp.clip(cx, 0.0, 19.0)
    colmod = (jax.lax.broadcasted_iota(jnp.int32, (S, F), 1) % 20).astype(jnp.float32)
    pf = (colmod >= thresh).astype(jnp.float32)

    gi = _dot_t(pf, wih_ref[...]) + bih_ref[...]
    gh = _dot_t(h, whh_ref[...]) + bhh_ref[...]
    i_r, i_z, i_n = gi[:, :G], gi[:, G:2 * G], gi[:, 2 * G:]
    h_r, h_z, h_n = gh[:, :G], gh[:, G:2 * G], gh[:, 2 * G:]
    r = jax.nn.sigmoid(i_r + h_r)
    z = jax.nn.sigmoid(i_z + h_z)
    n = jnp.tanh(i_n + r * h_n)
    hn = (1.0 - z) * n + z * h

    x1 = jax.nn.relu(_dot_t(hn, l1w_ref[...]) + l1b_ref[...])
    x2 = jax.nn.relu(_dot_t(x1, l2aw_ref[...]) + l2ab_ref[...])
    x2 = jax.nn.relu(_dot_t(x2, l2bw_ref[...]) + l2bb_ref[...])
    xn = _ln(x2, ln2g_ref[...], ln2b_ref[...])
    scores = jnp.sum(xn * sw_ref[...], axis=1, keepdims=True) + sb_ref[0, 0]

    smax = jnp.max(scores, axis=0, keepdims=True)
    e = jnp.exp(scores - smax)
    probs = e / jnp.sum(e, axis=0, keepdims=True)
    probs_ref[0] = probs

    bins = jnp.clip(jnp.round(mass_ref[0]), 0.0, float(_SPECT_BIN_N - 1))
    contrib = inten_ref[0] * probs
    lane = jax.lax.broadcasted_iota(jnp.int32, (S, _SPECT_BIN_N), 1).astype(jnp.float32)
    acc = jnp.zeros((S, _SPECT_BIN_N), jnp.float32)
    for m in range(M):
        onehot = (bins[:, m:m + 1] == lane).astype(jnp.float32)
        acc = acc + contrib[:, m:m + 1] * onehot
    spect_ref[0] = jnp.sum(acc, axis=0, keepdims=True)


def kernel(vert_feat_in, vert_mask_in, vert_element_oh, adj_oh, atom_subsets,
           atom_subsets_peaks, ln_g, ln_b, gru_w_ih, gru_w_hh, gru_b_ih,
           gru_b_hh, l1_w, l1_b, l2a_w, l2a_b, l2b_w, l2b_b, ln2_g, ln2_b,
           score_w, score_b):
    B, A, GF0, HW = vert_feat_in.shape
    G = GF0 * HW
    S = atom_subsets.shape[1]
    M = atom_subsets_peaks.shape[2]
    E = vert_element_oh.shape[2]
    F = int(np.sum(_FORMULA_OH_SIZES))
    D = l1_w.shape[0]

    x = vert_feat_in.reshape(B, A, G)
    mask3 = vert_mask_in.reshape(B, 1, A)
    elem_f = vert_element_oh.astype(jnp.float32)
    subs_f = atom_subsets.astype(jnp.float32)
    mass = atom_subsets_peaks[..., 0]
    inten = atom_subsets_peaks[..., 1]

    row = lambda v: v.reshape(1, -1)

    def bspec(shape, mapped=True):
        if mapped:
            return pl.BlockSpec(shape, lambda b: (b,) + (0,) * (len(shape) - 1))
        return pl.BlockSpec(shape, lambda b: (0,) * len(shape))

    in_specs = [
        bspec((1, A, G)),
        bspec((1, 1, A)),
        bspec((1, A, E)),
        bspec((1, S, A)),
        bspec((1, S, M)),
        bspec((1, S, M)),
        bspec((1, G), mapped=False),
        bspec((1, G), mapped=False),
        bspec((3 * G, F), mapped=False),
        bspec((3 * G, G), mapped=False),
        bspec((1, 3 * G), mapped=False),
        bspec((1, 3 * G), mapped=False),
        bspec((D, G), mapped=False),
        bspec((1, D), mapped=False),
        bspec((D, D), mapped=False),
        bspec((1, D), mapped=False),
        bspec((D, D), mapped=False),
        bspec((1, D), mapped=False),
        bspec((1, D), mapped=False),
        bspec((1, D), mapped=False),
        bspec((1, D), mapped=False),
        bspec((1, 1), mapped=False),
    ]
    out_specs = [
        bspec((1, 1, _SPECT_BIN_N)),
        bspec((1, S, 1)),
    ]
    spect3, probs3 = pl.pallas_call(
        _batch_kernel,
        grid=(B,),
        in_specs=in_specs,
        out_specs=out_specs,
        out_shape=[
            jax.ShapeDtypeStruct((B, 1, _SPECT_BIN_N), jnp.float32),
            jax.ShapeDtypeStruct((B, S, 1), jnp.float32),
        ],
    )(
        x, mask3, elem_f, subs_f, mass, inten,
        row(ln_g), row(ln_b),
        gru_w_ih, gru_w_hh, row(gru_b_ih), row(gru_b_hh),
        l1_w, row(l1_b), l2a_w, row(l2a_b), l2b_w, row(l2b_b),
        row(ln2_g), row(ln2_b), score_w, score_b.reshape(1, 1),
    )
    return spect3.reshape(B, _SPECT_BIN_N), probs3.reshape(B, S)


# speedup vs baseline: 3.1608x; 1.4612x over previous
import jax
import jax.numpy as jnp
import numpy as np
from jax.experimental import pallas as pl

_FORMULA_OH_SIZES = [20, 20, 20, 20, 20]
_SPECT_BIN_N = 512


def _dot_t(x, w):
    return jax.lax.dot_general(
        x, w, (((1,), (1,)), ((), ())), preferred_element_type=jnp.float32)


def _ln(x, g, b, eps=1e-5):
    m = jnp.mean(x, axis=-1, keepdims=True)
    v = jnp.mean((x - m) ** 2, axis=-1, keepdims=True)
    return (x - m) * jax.lax.rsqrt(v + eps) * g + b


def _full_kernel(
    x_ref,
    mask_ref,
    elem_ref,
    subs_ref,
    mass_ref,
    inten_ref,
    ln_g_ref, ln_b_ref,
    wih_ref,
    whh_ref,
    bih_ref, bhh_ref,
    l1w_ref,
    l1b_ref,
    l2aw_ref, l2ab_ref,
    l2bw_ref, l2bb_ref,
    ln2g_ref, ln2b_ref,
    sw_ref,
    sb_ref,
    spect_ref,
    probs_ref,
):
    B, S, A = subs_ref.shape
    G = x_ref.shape[2]
    M = mass_ref.shape[2]
    E = elem_ref.shape[2]
    F = 20 * E

    col = jax.lax.broadcasted_iota(jnp.int32, (E, F), 1)
    rowi = jax.lax.broadcasted_iota(jnp.int32, (E, F), 0)
    seg = (col // 20 == rowi).astype(jnp.float32)
    colmod = (jax.lax.broadcasted_iota(jnp.int32, (S, F), 1) % 20).astype(jnp.float32)

    h_rows = []
    pf_rows = []
    for b in range(B):
        x = x_ref[b]
        mask = mask_ref[b]
        subs_raw = subs_ref[b]
        subs_f = subs_raw * mask

        masked_x = x * mask.reshape(A, 1)
        swvs = jnp.dot(subs_f, masked_x, preferred_element_type=jnp.float32)
        size = jnp.sum(subs_f, axis=1, keepdims=True) + 0.0001
        h_rows.append(_ln(swvs / size, ln_g_ref[...], ln_b_ref[...]))

        p_mat = jnp.dot(elem_ref[b], seg, preferred_element_type=jnp.float32)
        cx = jnp.dot(subs_raw, p_mat, preferred_element_type=jnp.float32)
        thresh = jnp.clip(cx, 0.0, 19.0)
        pf_rows.append((colmod >= thresh).astype(jnp.float32))

    h = jnp.concatenate(h_rows, axis=0)
    pf = jnp.concatenate(pf_rows, axis=0)

    gi = _dot_t(pf, wih_ref[...]) + bih_ref[...]
    gh = _dot_t(h, whh_ref[...]) + bhh_ref[...]
    i_r, i_z, i_n = gi[:, :G], gi[:, G:2 * G], gi[:, 2 * G:]
    h_r, h_z, h_n = gh[:, :G], gh[:, G:2 * G], gh[:, 2 * G:]
    r = jax.nn.sigmoid(i_r + h_r)
    z = jax.nn.sigmoid(i_z + h_z)
    n = jnp.tanh(i_n + r * h_n)
    hn = (1.0 - z) * n + z * h

    x1 = jax.nn.relu(_dot_t(hn, l1w_ref[...]) + l1b_ref[...])
    x2 = jax.nn.relu(_dot_t(x1, l2aw_ref[...]) + l2ab_ref[...])
    x2 = jax.nn.relu(_dot_t(x2, l2bw_ref[...]) + l2bb_ref[...])
    xn = _ln(x2, ln2g_ref[...], ln2b_ref[...])
    scores = jnp.sum(xn * sw_ref[...], axis=1, keepdims=True) + sb_ref[0, 0]

    lane = jax.lax.broadcasted_iota(jnp.int32, (S, _SPECT_BIN_N), 1).astype(jnp.float32)
    for b in range(B):
        sc = scores[b * S:(b + 1) * S]
        smax = jnp.max(sc, axis=0, keepdims=True)
        e = jnp.exp(sc - smax)
        probs = e / jnp.sum(e, axis=0, keepdims=True)
        probs_ref[b] = probs

        bins = jnp.clip(jnp.round(mass_ref[b]), 0.0, float(_SPECT_BIN_N - 1))
        contrib = inten_ref[b] * probs
        acc = jnp.zeros((S, _SPECT_BIN_N), jnp.float32)
        for m in range(M):
            onehot = (bins[:, m:m + 1] == lane).astype(jnp.float32)
            acc = acc + contrib[:, m:m + 1] * onehot
        spect_ref[b] = jnp.sum(acc, axis=0, keepdims=True)


def kernel(vert_feat_in, vert_mask_in, vert_element_oh, adj_oh, atom_subsets,
           atom_subsets_peaks, ln_g, ln_b, gru_w_ih, gru_w_hh, gru_b_ih,
           gru_b_hh, l1_w, l1_b, l2a_w, l2a_b, l2b_w, l2b_b, ln2_g, ln2_b,
           score_w, score_b):
    B, A, GF0, HW = vert_feat_in.shape
    G = GF0 * HW
    S = atom_subsets.shape[1]
    M = atom_subsets_peaks.shape[2]
    E = vert_element_oh.shape[2]
    F = int(np.sum(_FORMULA_OH_SIZES))
    D = l1_w.shape[0]

    x = vert_feat_in.reshape(B, A, G)
    mask3 = vert_mask_in.reshape(B, 1, A)
    elem_f = vert_element_oh.astype(jnp.float32)
    subs_f = atom_subsets.astype(jnp.float32)
    mass = atom_subsets_peaks[..., 0]
    inten = atom_subsets_peaks[..., 1]

    row = lambda v: v.reshape(1, -1)

    spect3, probs3 = pl.pallas_call(
        _full_kernel,
        out_shape=[
            jax.ShapeDtypeStruct((B, 1, _SPECT_BIN_N), jnp.float32),
            jax.ShapeDtypeStruct((B, S, 1), jnp.float32),
        ],
    )(
        x, mask3, elem_f, subs_f, mass, inten,
        row(ln_g), row(ln_b),
        gru_w_ih, gru_w_hh, row(gru_b_ih), row(gru_b_hh),
        l1_w, row(l1_b), l2a_w, row(l2a_b), l2b_w, row(l2b_b),
        row(ln2_g), row(ln2_b), score_w, score_b.reshape(1, 1),
    )
    return spect3.reshape(B, _SPECT_BIN_N), probs3.reshape(B, S)
